# baseline (speedup 1.0000x reference)
import functools

import jax
import jax.numpy as jnp
from jax import lax
from jax.experimental import pallas as pl
from jax.experimental.pallas import tpu as pltpu
from jax.experimental.pallas import tpu_sc as plsc

_SPARSITY = 0.5

_N = 128 * 32768
_K = max(int(_SPARSITY * _N - 1), 0) + 1
_W = 32
_C = _N // _W
_SUB = 8192
_NG = (_C // _SUB) // 2

_mesh = plsc.VectorSubcoreMesh(core_axis_name="c", subcore_axis_name="s")
_SC_PARAMS = pltpu.CompilerParams(needs_layout_passes=False)


@functools.partial(
    pl.kernel,
    mesh=_mesh,
    out_type=jax.ShapeDtypeStruct((2, 256, 128), jnp.int32),
    scratch_types=[
        pltpu.VMEM((2, _SUB), jnp.float32),
        pltpu.VMEM((256, 128), jnp.int32),
        pltpu.VMEM((2, 128), jnp.int32),
        pltpu.VMEM_SHARED((256, 128), jnp.int32),
        pltpu.SemaphoreType.DMA,
        pltpu.SemaphoreType.DMA,
    ],
    compiler_params=_SC_PARAMS,
)
def _sc_pass1(x_hbm, h1_hbm, buf, h1, idxr, sh1, s0, s1):
    cid = lax.axis_index("c")
    sid = lax.axis_index("s")
    base = (sid * 2 + cid) * _C

    pltpu.make_async_copy(x_hbm.at[pl.ds(base, _SUB)], buf.at[0], s0).start()

    def zr(r, _):
        for cc in range(8):
            h1[r, pl.ds(cc * 16, 16)] = jnp.zeros((16,), jnp.int32)
        return 0

    lax.fori_loop(0, 256, zr, 0)
    for j in range(2):
        for cc in range(8):
            idxr[j, pl.ds(cc * 16, 16)] = lax.iota(jnp.int32, 16) + (j * 128 + cc * 16)

    @pl.when(sid == 0)
    def _():
        pltpu.sync_copy(h1, sh1)

    ones = jnp.ones((16,), jnp.int32)

    def process(b):
        def vbody(i, _):
            v = buf[b, pl.ds(i * 16, 16)]
            u = lax.bitcast_convert_type(v, jnp.int32) & jnp.int32(0x7FFFFFFF)
            row = lax.shift_right_logical(u, 23)
            col = lax.shift_right_logical(u, 16) & jnp.int32(127)
            plsc.addupdate_scatter(h1, [row, col], ones)
            return 0

        lax.fori_loop(0, _SUB // 16, vbody, 0)

    def g_body(g, _):
        pltpu.make_async_copy(
            x_hbm.at[pl.ds(base + (g * 2 + 1) * _SUB, _SUB)], buf.at[1], s1
        ).start()
        pltpu.make_async_copy(x_hbm.at[pl.ds(0, _SUB)], buf.at[0], s0).wait()
        process(0)

        @pl.when(g < _NG - 1)
        def _():
            pltpu.make_async_copy(
                x_hbm.at[pl.ds(base + (g * 2 + 2) * _SUB, _SUB)], buf.at[0], s0
            ).start()

        pltpu.make_async_copy(x_hbm.at[pl.ds(0, _SUB)], buf.at[1], s1).wait()
        process(1)
        return 0

    lax.fori_loop(0, _NG, g_body, 0)

    plsc.subcore_barrier()
    pltpu.sync_copy(h1.at[pl.ds(0, 128)], sh1.at[idxr.at[0]], add=True)
    pltpu.sync_copy(h1.at[pl.ds(128, 128)], sh1.at[idxr.at[1]], add=True)
    plsc.subcore_barrier()

    @pl.when(sid == 0)
    def _():
        pltpu.sync_copy(sh1, h1_hbm.at[cid])


def _prefix_rows(a):
    s = 1
    while s < a.shape[0]:
        pad = jnp.zeros((s, a.shape[1]), a.dtype)
        a = a + jnp.concatenate([pad, a[:-s]], axis=0)
        s *= 2
    return a


def _prefix_lanes(a):
    s = 1
    while s < a.shape[1]:
        pad = jnp.zeros((a.shape[0], s), a.dtype)
        a = a + jnp.concatenate([pad, a[:, :-s]], axis=1)
        s *= 2
    return a


def _flat_cum(h):
    incol = _prefix_lanes(h)
    rowtot = jnp.sum(h, axis=1, keepdims=True)
    rowcum = _prefix_rows(rowtot)
    return (rowcum - rowtot) + incol


def _tc_bin_kernel(h1_ref, o_ref):
    h = h1_ref[0] + h1_ref[1]
    cum = _flat_cum(h)
    b1 = jnp.sum((cum < _K).astype(jnp.int32))
    o_ref[...] = jnp.full((8, 128), b1, jnp.int32)


def _tc_select_bin(h1):
    return pl.pallas_call(
        _tc_bin_kernel,
        out_shape=jax.ShapeDtypeStruct((8, 128), jnp.int32),
    )(h1)


@functools.partial(
    pl.kernel,
    mesh=_mesh,
    out_type=jax.ShapeDtypeStruct((2, 512, 128), jnp.int32),
    scratch_types=[
        pltpu.VMEM((2, _SUB), jnp.float32),
        pltpu.VMEM((512, 128), jnp.int32),
        pltpu.VMEM((4, 128), jnp.int32),
        pltpu.VMEM((16,), jnp.int32),
        pltpu.VMEM_SHARED((512, 128), jnp.int32),
        pltpu.SemaphoreType.DMA,
        pltpu.SemaphoreType.DMA,
    ],
    compiler_params=_SC_PARAMS,
)
def _sc_pass2(x_hbm, b1_hbm, h2_hbm, buf, h2, idxr, b1v, sh2, s0, s1):
    cid = lax.axis_index("c")
    sid = lax.axis_index("s")
    base = (sid * 2 + cid) * _C

    pltpu.make_async_copy(x_hbm.at[pl.ds(base, _SUB)], buf.at[0], s0).start()
    pltpu.sync_copy(b1_hbm.at[0, pl.ds(0, 16)], b1v)

    def zr(r, _):
        for cc in range(8):
            h2[r, pl.ds(cc * 16, 16)] = jnp.zeros((16,), jnp.int32)
        return 0

    lax.fori_loop(0, 512, zr, 0)
    for j in range(4):
        for cc in range(8):
            idxr[j, pl.ds(cc * 16, 16)] = lax.iota(jnp.int32, 16) + (j * 128 + cc * 16)

    @pl.when(sid == 0)
    def _():
        pltpu.sync_copy(h2, sh2)

    ones = jnp.ones((16,), jnp.int32)
    vb1 = b1v[...]

    def process(b):
        def vbody(i, _):
            v = buf[b, pl.ds(i * 16, 16)]
            u = lax.bitcast_convert_type(v, jnp.int32) & jnp.int32(0x7FFFFFFF)
            m = lax.shift_right_logical(u, 16) == vb1
            row = lax.shift_right_logical(u, 7) & jnp.int32(511)
            col = u & jnp.int32(127)
            plsc.addupdate_scatter(h2, [row, col], ones, mask=m)
            return 0

        lax.fori_loop(0, _SUB // 16, vbody, 0)

    def g_body(g, _):
        pltpu.make_async_copy(
            x_hbm.at[pl.ds(base + (g * 2 + 1) * _SUB, _SUB)], buf.at[1], s1
        ).start()
        pltpu.make_async_copy(x_hbm.at[pl.ds(0, _SUB)], buf.at[0], s0).wait()
        process(0)

        @pl.when(g < _NG - 1)
        def _():
            pltpu.make_async_copy(
                x_hbm.at[pl.ds(base + (g * 2 + 2) * _SUB, _SUB)], buf.at[0], s0
            ).start()

        pltpu.make_async_copy(x_hbm.at[pl.ds(0, _SUB)], buf.at[1], s1).wait()
        process(1)
        return 0

    lax.fori_loop(0, _NG, g_body, 0)

    plsc.subcore_barrier()
    for q in range(4):
        pltpu.sync_copy(h2.at[pl.ds(q * 128, 128)], sh2.at[idxr.at[q]], add=True)
    plsc.subcore_barrier()

    @pl.when(sid == 0)
    def _():
        pltpu.sync_copy(sh2, h2_hbm.at[cid])


def _tc_apply_kernel(x_ref, h1_ref, h2_ref, o_ref, t_smem):
    pid = pl.program_id(0)

    @pl.when(pid == 0)
    def _():
        h = h1_ref[0] + h1_ref[1]
        cum = _flat_cum(h)
        lt = cum < _K
        b1 = jnp.sum(lt.astype(jnp.int32))
        below = jnp.sum(h * lt.astype(jnp.int32))
        kp = _K - below
        h2 = h2_ref[0] + h2_ref[1]
        cum2 = _flat_cum(h2)
        low = jnp.sum((cum2 < kp).astype(jnp.int32))
        t_smem[0] = (b1 << 16) | low

    t = t_smem[0]
    xb = x_ref[...]
    u = lax.bitcast_convert_type(xb, jnp.int32) & jnp.int32(0x7FFFFFFF)
    o_ref[...] = jnp.where(u >= t, xb, jnp.float32(0.0))


def _tc_apply(x, h1, h2):
    return pl.pallas_call(
        _tc_apply_kernel,
        grid=(x.shape[0] // 8,),
        in_specs=[
            pl.BlockSpec((8, x.shape[1]), lambda i: (i, 0)),
            pl.BlockSpec((2, 256, 128), lambda i: (0, 0, 0)),
            pl.BlockSpec((2, 512, 128), lambda i: (0, 0, 0)),
        ],
        out_specs=pl.BlockSpec((8, x.shape[1]), lambda i: (i, 0)),
        out_shape=jax.ShapeDtypeStruct(x.shape, x.dtype),
        scratch_shapes=[pltpu.SMEM((1,), jnp.int32)],
    )(x, h1, h2)


def kernel(x, mask):
    del mask
    assert x.size == _N and x.dtype == jnp.float32
    xf = x.reshape(-1)
    h1 = _sc_pass1(xf)
    b1v = _tc_select_bin(h1)
    h2 = _sc_pass2(xf, b1v)
    return _tc_apply(x, h1, h2)


# speedup vs baseline: 2.1181x; 2.1181x over previous
import functools

import jax
import jax.numpy as jnp
from jax import lax
from jax.experimental import pallas as pl
from jax.experimental.pallas import tpu as pltpu
from jax.experimental.pallas import tpu_sc as plsc

_SPARSITY = 0.5

_N = 128 * 32768
_K = max(int(_SPARSITY * _N - 1), 0) + 1
_W = 32
_C = _N // _W
_SUB = 8192
_NG = (_C // _SUB) // 2

_mesh = plsc.VectorSubcoreMesh(core_axis_name="c", subcore_axis_name="s")
_SC_PARAMS = pltpu.CompilerParams(needs_layout_passes=False)


@functools.partial(
    pl.kernel,
    mesh=_mesh,
    out_type=jax.ShapeDtypeStruct((2, 256, 128), jnp.int32),
    scratch_types=[
        pltpu.VMEM((2, _SUB), jnp.float32),
        pltpu.VMEM((256, 128), jnp.int32),
        pltpu.VMEM((2, 128), jnp.int32),
        pltpu.VMEM_SHARED((256, 128), jnp.int32),
        pltpu.SemaphoreType.DMA,
        pltpu.SemaphoreType.DMA,
    ],
    compiler_params=_SC_PARAMS,
)
def _sc_pass1(x_hbm, h1_hbm, buf, h1, idxr, sh1, s0, s1):
    cid = lax.axis_index("c")
    sid = lax.axis_index("s")
    base = (sid * 2 + cid) * _C

    pltpu.make_async_copy(x_hbm.at[pl.ds(base, _SUB)], buf.at[0], s0).start()

    @plsc.parallel_loop(0, 256, 1, unroll=4)
    def _(r):
        for cc in range(8):
            h1[r, pl.ds(cc * 16, 16)] = jnp.zeros((16,), jnp.int32)

    for j in range(2):
        for cc in range(8):
            idxr[j, pl.ds(cc * 16, 16)] = lax.iota(jnp.int32, 16) + (j * 128 + cc * 16)

    @pl.when(sid == 0)
    def _():
        pltpu.sync_copy(h1, sh1)

    ones = jnp.ones((16,), jnp.int32)

    def process(b):
        @plsc.parallel_loop(0, _SUB // 16, 1, unroll=8)
        def _(i):
            v = buf[b, pl.ds(i * 16, 16)]
            u = lax.bitcast_convert_type(v, jnp.int32) & jnp.int32(0x7FFFFFFF)
            row = lax.shift_right_logical(u, 23)
            col = lax.shift_right_logical(u, 16) & jnp.int32(127)
            plsc.addupdate_scatter(h1, [row, col], ones)

    def g_body(g, _):
        pltpu.make_async_copy(
            x_hbm.at[pl.ds(base + (g * 2 + 1) * _SUB, _SUB)], buf.at[1], s1
        ).start()
        pltpu.make_async_copy(x_hbm.at[pl.ds(0, _SUB)], buf.at[0], s0).wait()
        process(0)

        @pl.when(g < _NG - 1)
        def _():
            pltpu.make_async_copy(
                x_hbm.at[pl.ds(base + (g * 2 + 2) * _SUB, _SUB)], buf.at[0], s0
            ).start()

        pltpu.make_async_copy(x_hbm.at[pl.ds(0, _SUB)], buf.at[1], s1).wait()
        process(1)
        return 0

    lax.fori_loop(0, _NG, g_body, 0)

    plsc.subcore_barrier()
    pltpu.sync_copy(h1.at[pl.ds(0, 128)], sh1.at[idxr.at[0]], add=True)
    pltpu.sync_copy(h1.at[pl.ds(128, 128)], sh1.at[idxr.at[1]], add=True)
    plsc.subcore_barrier()

    @pl.when(sid == 0)
    def _():
        pltpu.sync_copy(sh1, h1_hbm.at[cid])


def _prefix_rows(a):
    s = 1
    while s < a.shape[0]:
        pad = jnp.zeros((s, a.shape[1]), a.dtype)
        a = a + jnp.concatenate([pad, a[:-s]], axis=0)
        s *= 2
    return a


def _prefix_lanes(a):
    s = 1
    while s < a.shape[1]:
        pad = jnp.zeros((a.shape[0], s), a.dtype)
        a = a + jnp.concatenate([pad, a[:, :-s]], axis=1)
        s *= 2
    return a


def _flat_cum(h):
    incol = _prefix_lanes(h)
    rowtot = jnp.sum(h, axis=1, keepdims=True)
    rowcum = _prefix_rows(rowtot)
    return (rowcum - rowtot) + incol


def _tc_bin_kernel(h1_ref, o_ref):
    h = h1_ref[0] + h1_ref[1]
    cum = _flat_cum(h)
    b1 = jnp.sum((cum < _K).astype(jnp.int32))
    o_ref[...] = jnp.full((8, 128), b1, jnp.int32)


def _tc_select_bin(h1):
    return pl.pallas_call(
        _tc_bin_kernel,
        out_shape=jax.ShapeDtypeStruct((8, 128), jnp.int32),
    )(h1)


@functools.partial(
    pl.kernel,
    mesh=_mesh,
    out_type=jax.ShapeDtypeStruct((2, 512, 128), jnp.int32),
    scratch_types=[
        pltpu.VMEM((2, _SUB), jnp.float32),
        pltpu.VMEM((512, 128), jnp.int32),
        pltpu.VMEM((4, 128), jnp.int32),
        pltpu.VMEM((16,), jnp.int32),
        pltpu.VMEM_SHARED((512, 128), jnp.int32),
        pltpu.SemaphoreType.DMA,
        pltpu.SemaphoreType.DMA,
    ],
    compiler_params=_SC_PARAMS,
)
def _sc_pass2(x_hbm, b1_hbm, h2_hbm, buf, h2, idxr, b1v, sh2, s0, s1):
    cid = lax.axis_index("c")
    sid = lax.axis_index("s")
    base = (sid * 2 + cid) * _C

    pltpu.make_async_copy(x_hbm.at[pl.ds(base, _SUB)], buf.at[0], s0).start()
    pltpu.sync_copy(b1_hbm.at[0, pl.ds(0, 16)], b1v)

    @plsc.parallel_loop(0, 512, 1, unroll=4)
    def _(r):
        for cc in range(8):
            h2[r, pl.ds(cc * 16, 16)] = jnp.zeros((16,), jnp.int32)

    for j in range(4):
        for cc in range(8):
            idxr[j, pl.ds(cc * 16, 16)] = lax.iota(jnp.int32, 16) + (j * 128 + cc * 16)

    @pl.when(sid == 0)
    def _():
        pltpu.sync_copy(h2, sh2)

    ones = jnp.ones((16,), jnp.int32)
    vb1 = b1v[...]

    def process(b):
        @plsc.parallel_loop(0, _SUB // 16, 1, unroll=8)
        def _(i):
            v = buf[b, pl.ds(i * 16, 16)]
            u = lax.bitcast_convert_type(v, jnp.int32) & jnp.int32(0x7FFFFFFF)
            m = lax.shift_right_logical(u, 16) == vb1
            row = lax.shift_right_logical(u, 7) & jnp.int32(511)
            col = u & jnp.int32(127)
            plsc.addupdate_scatter(h2, [row, col], ones, mask=m)

    def g_body(g, _):
        pltpu.make_async_copy(
            x_hbm.at[pl.ds(base + (g * 2 + 1) * _SUB, _SUB)], buf.at[1], s1
        ).start()
        pltpu.make_async_copy(x_hbm.at[pl.ds(0, _SUB)], buf.at[0], s0).wait()
        process(0)

        @pl.when(g < _NG - 1)
        def _():
            pltpu.make_async_copy(
                x_hbm.at[pl.ds(base + (g * 2 + 2) * _SUB, _SUB)], buf.at[0], s0
            ).start()

        pltpu.make_async_copy(x_hbm.at[pl.ds(0, _SUB)], buf.at[1], s1).wait()
        process(1)
        return 0

    lax.fori_loop(0, _NG, g_body, 0)

    plsc.subcore_barrier()
    for q in range(4):
        pltpu.sync_copy(h2.at[pl.ds(q * 128, 128)], sh2.at[idxr.at[q]], add=True)
    plsc.subcore_barrier()

    @pl.when(sid == 0)
    def _():
        pltpu.sync_copy(sh2, h2_hbm.at[cid])


def _tc_apply_kernel(x_ref, h1_ref, h2_ref, o_ref, t_smem):
    pid = pl.program_id(0)

    @pl.when(pid == 0)
    def _():
        h = h1_ref[0] + h1_ref[1]
        cum = _flat_cum(h)
        lt = cum < _K
        b1 = jnp.sum(lt.astype(jnp.int32))
        below = jnp.sum(h * lt.astype(jnp.int32))
        kp = _K - below
        h2 = h2_ref[0] + h2_ref[1]
        cum2 = _flat_cum(h2)
        low = jnp.sum((cum2 < kp).astype(jnp.int32))
        t_smem[0] = (b1 << 16) | low

    t = t_smem[0]
    xb = x_ref[...]
    u = lax.bitcast_convert_type(xb, jnp.int32) & jnp.int32(0x7FFFFFFF)
    o_ref[...] = jnp.where(u >= t, xb, jnp.float32(0.0))


def _tc_apply(x, h1, h2):
    return pl.pallas_call(
        _tc_apply_kernel,
        grid=(x.shape[0] // 8,),
        in_specs=[
            pl.BlockSpec((8, x.shape[1]), lambda i: (i, 0)),
            pl.BlockSpec((2, 256, 128), lambda i: (0, 0, 0)),
            pl.BlockSpec((2, 512, 128), lambda i: (0, 0, 0)),
        ],
        out_specs=pl.BlockSpec((8, x.shape[1]), lambda i: (i, 0)),
        out_shape=jax.ShapeDtypeStruct(x.shape, x.dtype),
        scratch_shapes=[pltpu.SMEM((1,), jnp.int32)],
    )(x, h1, h2)


def kernel(x, mask):
    del mask
    assert x.size == _N and x.dtype == jnp.float32
    xf = x.reshape(-1)
    h1 = _sc_pass1(xf)
    b1v = _tc_select_bin(h1)
    h2 = _sc_pass2(xf, b1v)
    return _tc_apply(x, h1, h2)


# speedup vs baseline: 2.5306x; 1.1948x over previous
import functools

import jax
import jax.numpy as jnp
from jax import lax
from jax.experimental import pallas as pl
from jax.experimental.pallas import tpu as pltpu
from jax.experimental.pallas import tpu_sc as plsc

_SPARSITY = 0.5

_N = 128 * 32768
_K = max(int(_SPARSITY * _N - 1), 0) + 1
_W = 32
_C = _N // _W
_ROWS_PER_W = 128 // _W
_SUB = 8192
_NG = (_C // _SUB) // 2

_mesh = plsc.VectorSubcoreMesh(core_axis_name="c", subcore_axis_name="s")
_SC_PARAMS = pltpu.CompilerParams(needs_layout_passes=False)


@functools.partial(
    pl.kernel,
    mesh=_mesh,
    out_type=jax.ShapeDtypeStruct((2, 256, 128), jnp.int32),
    scratch_types=[
        pltpu.VMEM((2, _SUB), jnp.float32),
        pltpu.VMEM((256, 128), jnp.int32),
        pltpu.VMEM((2, 128), jnp.int32),
        pltpu.VMEM_SHARED((256, 128), jnp.int32),
        pltpu.SemaphoreType.DMA,
        pltpu.SemaphoreType.DMA,
    ],
    compiler_params=_SC_PARAMS,
)
def _sc_pass1(x_hbm, h1_hbm, buf, h1, idxr, sh1, s0, s1):
    cid = lax.axis_index("c")
    sid = lax.axis_index("s")
    row0 = (sid * 2 + cid) * _ROWS_PER_W

    def chunk(j):
        return x_hbm.at[row0 + lax.shift_right_logical(j, 2), pl.ds((j & 3) * _SUB, _SUB)]

    pltpu.make_async_copy(chunk(jnp.int32(0)), buf.at[0], s0).start()

    @plsc.parallel_loop(0, 256, 1, unroll=4)
    def _(r):
        for cc in range(8):
            h1[r, pl.ds(cc * 16, 16)] = jnp.zeros((16,), jnp.int32)

    for j in range(2):
        for cc in range(8):
            idxr[j, pl.ds(cc * 16, 16)] = lax.iota(jnp.int32, 16) + (j * 128 + cc * 16)

    @pl.when(sid == 0)
    def _():
        pltpu.sync_copy(h1, sh1)

    ones = jnp.ones((16,), jnp.int32)

    def process(b):
        @plsc.parallel_loop(0, _SUB // 16, 1, unroll=8)
        def _(i):
            v = buf[b, pl.ds(i * 16, 16)]
            u = lax.bitcast_convert_type(v, jnp.int32) & jnp.int32(0x7FFFFFFF)
            row = lax.shift_right_logical(u, 23)
            col = lax.shift_right_logical(u, 16) & jnp.int32(127)
            plsc.addupdate_scatter(h1, [row, col], ones)

    def g_body(g, _):
        pltpu.make_async_copy(chunk(g * 2 + 1), buf.at[1], s1).start()
        pltpu.make_async_copy(x_hbm.at[row0, pl.ds(0, _SUB)], buf.at[0], s0).wait()
        process(0)

        @pl.when(g < _NG - 1)
        def _():
            pltpu.make_async_copy(chunk(g * 2 + 2), buf.at[0], s0).start()

        pltpu.make_async_copy(x_hbm.at[row0, pl.ds(0, _SUB)], buf.at[1], s1).wait()
        process(1)
        return 0

    lax.fori_loop(0, _NG, g_body, 0)

    plsc.subcore_barrier()
    pltpu.sync_copy(h1.at[pl.ds(0, 128)], sh1.at[idxr.at[0]], add=True)
    pltpu.sync_copy(h1.at[pl.ds(128, 128)], sh1.at[idxr.at[1]], add=True)
    plsc.subcore_barrier()

    @pl.when(sid == 0)
    def _():
        pltpu.sync_copy(sh1, h1_hbm.at[cid])


def _prefix_rows(a):
    s = 1
    while s < a.shape[0]:
        pad = jnp.zeros((s, a.shape[1]), a.dtype)
        a = a + jnp.concatenate([pad, a[:-s]], axis=0)
        s *= 2
    return a


def _prefix_lanes(a):
    s = 1
    while s < a.shape[1]:
        pad = jnp.zeros((a.shape[0], s), a.dtype)
        a = a + jnp.concatenate([pad, a[:, :-s]], axis=1)
        s *= 2
    return a


def _flat_cum(h):
    incol = _prefix_lanes(h)
    rowtot = jnp.sum(h, axis=1, keepdims=True)
    rowcum = _prefix_rows(rowtot)
    return (rowcum - rowtot) + incol


def _tc_bin_kernel(h1_ref, o_ref):
    h = h1_ref[0] + h1_ref[1]
    cum = _flat_cum(h)
    b1 = jnp.sum((cum < _K).astype(jnp.int32))
    o_ref[...] = jnp.full((8, 128), b1, jnp.int32)


def _tc_select_bin(h1):
    return pl.pallas_call(
        _tc_bin_kernel,
        out_shape=jax.ShapeDtypeStruct((8, 128), jnp.int32),
    )(h1)


@functools.partial(
    pl.kernel,
    mesh=_mesh,
    out_type=jax.ShapeDtypeStruct((2, 512, 128), jnp.int32),
    scratch_types=[
        pltpu.VMEM((2, _SUB), jnp.float32),
        pltpu.VMEM((512, 128), jnp.int32),
        pltpu.VMEM((4, 128), jnp.int32),
        pltpu.VMEM((16,), jnp.int32),
        pltpu.VMEM_SHARED((512, 128), jnp.int32),
        pltpu.SemaphoreType.DMA,
        pltpu.SemaphoreType.DMA,
    ],
    compiler_params=_SC_PARAMS,
)
def _sc_pass2(x_hbm, b1_hbm, h2_hbm, buf, h2, idxr, b1v, sh2, s0, s1):
    cid = lax.axis_index("c")
    sid = lax.axis_index("s")
    row0 = (sid * 2 + cid) * _ROWS_PER_W

    def chunk(j):
        return x_hbm.at[row0 + lax.shift_right_logical(j, 2), pl.ds((j & 3) * _SUB, _SUB)]

    pltpu.make_async_copy(chunk(jnp.int32(0)), buf.at[0], s0).start()
    pltpu.sync_copy(b1_hbm.at[0, pl.ds(0, 16)], b1v)

    @plsc.parallel_loop(0, 512, 1, unroll=4)
    def _(r):
        for cc in range(8):
            h2[r, pl.ds(cc * 16, 16)] = jnp.zeros((16,), jnp.int32)

    for j in range(4):
        for cc in range(8):
            idxr[j, pl.ds(cc * 16, 16)] = lax.iota(jnp.int32, 16) + (j * 128 + cc * 16)

    @pl.when(sid == 0)
    def _():
        pltpu.sync_copy(h2, sh2)

    ones = jnp.ones((16,), jnp.int32)
    vb1 = b1v[...]

    def process(b):
        @plsc.parallel_loop(0, _SUB // 16, 1, unroll=8)
        def _(i):
            v = buf[b, pl.ds(i * 16, 16)]
            u = lax.bitcast_convert_type(v, jnp.int32) & jnp.int32(0x7FFFFFFF)
            m = lax.shift_right_logical(u, 16) == vb1
            row = lax.shift_right_logical(u, 7) & jnp.int32(511)
            col = u & jnp.int32(127)
            plsc.addupdate_scatter(h2, [row, col], ones, mask=m)

    def g_body(g, _):
        pltpu.make_async_copy(chunk(g * 2 + 1), buf.at[1], s1).start()
        pltpu.make_async_copy(x_hbm.at[row0, pl.ds(0, _SUB)], buf.at[0], s0).wait()
        process(0)

        @pl.when(g < _NG - 1)
        def _():
            pltpu.make_async_copy(chunk(g * 2 + 2), buf.at[0], s0).start()

        pltpu.make_async_copy(x_hbm.at[row0, pl.ds(0, _SUB)], buf.at[1], s1).wait()
        process(1)
        return 0

    lax.fori_loop(0, _NG, g_body, 0)

    plsc.subcore_barrier()
    for q in range(4):
        pltpu.sync_copy(h2.at[pl.ds(q * 128, 128)], sh2.at[idxr.at[q]], add=True)
    plsc.subcore_barrier()

    @pl.when(sid == 0)
    def _():
        pltpu.sync_copy(sh2, h2_hbm.at[cid])


def _tc_apply_kernel(x_ref, h1_ref, h2_ref, o_ref, t_smem):
    pid = pl.program_id(0)

    @pl.when(pid == 0)
    def _():
        h = h1_ref[0] + h1_ref[1]
        cum = _flat_cum(h)
        lt = cum < _K
        b1 = jnp.sum(lt.astype(jnp.int32))
        below = jnp.sum(h * lt.astype(jnp.int32))
        kp = _K - below
        h2 = h2_ref[0] + h2_ref[1]
        cum2 = _flat_cum(h2)
        low = jnp.sum((cum2 < kp).astype(jnp.int32))
        t_smem[0] = (b1 << 16) | low

    t = t_smem[0]
    xb = x_ref[...]
    u = lax.bitcast_convert_type(xb, jnp.int32) & jnp.int32(0x7FFFFFFF)
    o_ref[...] = jnp.where(u >= t, xb, jnp.float32(0.0))


def _tc_apply(x, h1, h2):
    return pl.pallas_call(
        _tc_apply_kernel,
        grid=(x.shape[0] // 8,),
        in_specs=[
            pl.BlockSpec((8, x.shape[1]), lambda i: (i, 0)),
            pl.BlockSpec((2, 256, 128), lambda i: (0, 0, 0)),
            pl.BlockSpec((2, 512, 128), lambda i: (0, 0, 0)),
        ],
        out_specs=pl.BlockSpec((8, x.shape[1]), lambda i: (i, 0)),
        out_shape=jax.ShapeDtypeStruct(x.shape, x.dtype),
        scratch_shapes=[pltpu.SMEM((1,), jnp.int32)],
    )(x, h1, h2)


def kernel(x, mask):
    del mask
    assert x.shape == (128, 32768) and x.dtype == jnp.float32
    h1 = _sc_pass1(x)
    b1v = _tc_select_bin(h1)
    h2 = _sc_pass2(x, b1v)
    return _tc_apply(x, h1, h2)
